# baseline (speedup 1.0000x reference)
import functools

import jax
import jax.numpy as jnp
from jax import lax
from jax.experimental import pallas as pl
from jax.experimental.pallas import tpu as pltpu
from jax.experimental.pallas import tpu_sc as plsc

NC = 2
NS = 16
NW = NC * NS
LANES = 16

B = 1024
L = 200
EMB = 64
N = B * L
PER_W = N // NW
CHUNK = L
NCHUNK = PER_W // CHUNK
COLV = EMB // LANES


def _body(idx_hbm, table_hbm, pos_hbm, out_hbm, idx_v, pos_v, rows_v, sem):
    wid = lax.axis_index("s") * NC + lax.axis_index("c")
    base = wid * PER_W

    pltpu.sync_copy(idx_hbm.at[pl.ds(base, PER_W)], idx_v)
    pltpu.sync_copy(pos_hbm, pos_v)

    def chunk_body(k, _):
        row0 = base + k * CHUNK
        pltpu.async_copy(
            table_hbm.at[idx_v.at[pl.ds(k * CHUNK, CHUNK)]], rows_v, sem
        ).wait()

        def add_row(r, _):
            for c in range(COLV):
                sl = pl.ds(c * LANES, LANES)
                rows_v[r, sl] = rows_v[r, sl] + pos_v[r, sl]
            return 0

        lax.fori_loop(0, CHUNK, add_row, 0)

        pltpu.sync_copy(rows_v, out_hbm.at[pl.ds(row0, CHUNK)])
        return 0

    lax.fori_loop(0, NCHUNK, chunk_body, 0)


@jax.jit
def _embed(x_flat, W, pos_emb):
    mesh = plsc.VectorSubcoreMesh(core_axis_name="c", subcore_axis_name="s")
    k = pl.kernel(
        _body,
        out_type=jax.ShapeDtypeStruct((N, EMB), jnp.float32),
        mesh=mesh,
        scratch_types=[
            pltpu.VMEM((PER_W,), jnp.int32),
            pltpu.VMEM((L, EMB), jnp.float32),
            pltpu.VMEM((CHUNK, EMB), jnp.float32),
            pltpu.SemaphoreType.DMA,
        ],
        compiler_params=pltpu.CompilerParams(use_tc_tiling_on_sc=False),
    )
    return k(x_flat, W, pos_emb)


def kernel(x, W, pos_emb):
    x_flat = x.reshape(-1).astype(jnp.int32)
    out = _embed(x_flat, W, pos_emb[:L])
    return out.reshape(x.shape[0], x.shape[1], EMB)


# speedup vs baseline: 1.1416x; 1.1416x over previous
import functools

import jax
import jax.numpy as jnp
from jax import lax
from jax.experimental import pallas as pl
from jax.experimental.pallas import tpu as pltpu
from jax.experimental.pallas import tpu_sc as plsc

NC = 2
NS = 16
NW = NC * NS
LANES = 16

B = 1024
L = 200
EMB = 64
EMBP = 128
N = B * L
PER_W = N // NW
CHUNK = L
NCHUNK = PER_W // CHUNK
COLV = EMB // LANES


def _body(idx_hbm, table_hbm, pos_hbm, out_hbm, idx_v, pos_v, rows_v, sem):
    wid = lax.axis_index("s") * NC + lax.axis_index("c")
    base = wid * PER_W

    pltpu.sync_copy(idx_hbm.at[pl.ds(base, PER_W)], idx_v)
    pltpu.sync_copy(pos_hbm, pos_v)

    def chunk_body(k, _):
        row0 = base + k * CHUNK
        pltpu.async_copy(
            table_hbm.at[idx_v.at[pl.ds(k * CHUNK, CHUNK)]], rows_v, sem
        ).wait()

        def add_row(r, _):
            for c in range(COLV):
                sl = pl.ds(c * LANES, LANES)
                rows_v[r, sl] = rows_v[r, sl] + pos_v[r, sl]
            return 0

        lax.fori_loop(0, CHUNK, add_row, 0)

        pltpu.sync_copy(rows_v, out_hbm.at[pl.ds(row0, CHUNK)])
        return 0

    lax.fori_loop(0, NCHUNK, chunk_body, 0)


@jax.jit
def _embed(x_flat, W_pad, pos_pad):
    mesh = plsc.VectorSubcoreMesh(core_axis_name="c", subcore_axis_name="s")
    k = pl.kernel(
        _body,
        out_type=jax.ShapeDtypeStruct((N, EMBP), jnp.float32),
        mesh=mesh,
        scratch_types=[
            pltpu.VMEM((PER_W,), jnp.int32),
            pltpu.VMEM((L, EMBP), jnp.float32),
            pltpu.VMEM((CHUNK, EMBP), jnp.float32),
            pltpu.SemaphoreType.DMA,
        ],
        compiler_params=pltpu.CompilerParams(use_tc_tiling_on_sc=True),
    )
    return k(x_flat, W_pad, pos_pad)


def kernel(x, W, pos_emb):
    x_flat = x.reshape(-1).astype(jnp.int32)
    W_pad = jnp.pad(W, ((0, 0), (0, EMBP - EMB)))
    pos_pad = jnp.pad(pos_emb[:L], ((0, 0), (0, EMBP - EMB)))
    out = _embed(x_flat, W_pad, pos_pad)
    return out[:, :EMB].reshape(x.shape[0], x.shape[1], EMB)
